# Initial kernel scaffold; baseline (speedup 1.0000x reference)
import jax
import jax.numpy as jnp
from jax.experimental import pallas as pl


def kernel(GS_feats, scene_bounds):
    raise NotImplementedError("write your pallas kernel here")


# speedup vs baseline: 1.1152x; 1.1152x over previous
import functools

import jax
import jax.numpy as jnp
from jax import lax
from jax.experimental import pallas as pl
from jax.experimental.pallas import tpu as pltpu
from jax.experimental.pallas import tpu_sc as plsc

GRID = 128
CELLS = GRID * GRID
C = 196
CP = 208
HALF = CP // 2
NPLANES = 3
NSUB = 16
PCHUNK = 128


def _pre_body(consts_ref, gr_ref, p_ref, w_ref, idx_ref):
    Bn = gr_ref.shape[2]
    npts = pl.num_programs(1) * Bn
    rmod = npts % C
    n0 = pl.program_id(1) * Bn
    a = gr_ref[0]
    ci = lax.broadcasted_iota(jnp.int32, (C, Bn), 0)
    ni = n0 + lax.broadcasted_iota(jnp.int32, (C, Bn), 1)
    q = lax.rem(ni + rmod * ci, C)
    s0 = consts_ref[0]
    o0 = consts_ref[1]
    s1 = consts_ref[2]
    o1 = consts_ref[3]
    s2 = consts_ref[4]
    o2 = consts_ref[5]
    bv = jnp.where(q == 0, a * s0 + o0,
         jnp.where(q == 1, a * s1 + o1,
         jnp.where(q == 2, a * s2 + o2, a)))
    p4 = p_ref[0]
    alpha = jax.nn.sigmoid(p4[3:4, :])
    w_ref[0, 0:C, :] = bv * alpha
    w_ref[0, C:C + 1, :] = alpha
    w_ref[0, C + 1:CP, :] = jnp.zeros((CP - C - 1, Bn), jnp.float32)

    x = p4[0:1, :] * s0 + o0
    y = p4[1:2, :] * s1 + o1
    z = p4[2:3, :] * s2 + o2

    def cell(u):
        g = ((u * 0.5 + 0.5) * (GRID - 1)).astype(jnp.int32)
        return jnp.clip(g, 0, GRID - 1)

    gx, gy, gz = cell(x), cell(y), cell(z)
    idx_ref[0, 0:1, :] = gx * GRID + gy
    idx_ref[0, 1:2, :] = gx * GRID + gz
    idx_ref[0, 2:3, :] = gy * GRID + gz


def _post_body(a0_ref, a1_ref, o0_ref, o1_ref):
    m0 = a0_ref[0]
    m1 = a1_ref[0]
    w = jnp.maximum(m1[:, C - HALF:C - HALF + 1], 1e-6)
    o0_ref[0] = m0 / w
    o1_ref[0] = m1[:, 0:C - HALF] / w


def _make_sc_scatter(nbatch, npts):
    pts_per_sub = npts // NSUB
    nchunks = pts_per_sub // PCHUNK
    rows_per_sub = CELLS // NSUB
    nrounds = nbatch * NPLANES
    mesh = plsc.VectorSubcoreMesh(core_axis_name="c", subcore_axis_name="s")

    @functools.partial(
        pl.kernel,
        mesh=mesh,
        compiler_params=pltpu.CompilerParams(use_tc_tiling_on_sc=False),
        out_type=jax.ShapeDtypeStruct((2, nbatch, NPLANES, CELLS, HALF),
                                      jnp.float32),
        scratch_types=[
            pltpu.VMEM_SHARED((CELLS, HALF), jnp.float32),
            pltpu.VMEM((PCHUNK,), jnp.int32),
            pltpu.VMEM((PCHUNK, HALF), jnp.float32),
        ],
    )
    def sc_scatter(wt_hbm, idx_hbm, z_hbm, out_hbm, acc, idxv, rows):
        cid = lax.axis_index("c")
        sid = lax.axis_index("s")
        r0 = sid * rows_per_sub
        pbase = sid * pts_per_sub

        def round_body(r, carry):
            b = r // NPLANES
            p = lax.rem(r, NPLANES)
            pltpu.sync_copy(z_hbm.at[pl.ds(r0, rows_per_sub), :],
                            acc.at[pl.ds(r0, rows_per_sub), :])
            plsc.subcore_barrier()

            def chunk(k, carry2):
                base = pbase + k * PCHUNK
                pltpu.sync_copy(idx_hbm.at[b, p, 0, pl.ds(base, PCHUNK)],
                                idxv)
                pltpu.sync_copy(wt_hbm.at[cid, b, pl.ds(base, PCHUNK), :],
                                rows)
                pltpu.sync_copy(rows, acc.at[idxv], add=True)
                return carry2

            lax.fori_loop(0, nchunks, chunk, 0)
            plsc.subcore_barrier()
            pltpu.sync_copy(
                acc.at[pl.ds(r0, rows_per_sub), :],
                out_hbm.at[cid, b, p, pl.ds(r0, rows_per_sub), :])
            return carry

        lax.fori_loop(0, nrounds, round_body, 0)

    return sc_scatter


def kernel(GS_feats, scene_bounds):
    nbatch, npts, nchan = GS_feats.shape
    sb = scene_bounds.astype(jnp.float32)
    s0 = 2.0 / (sb[1] - sb[0])
    o0 = -2.0 * sb[0] / (sb[1] - sb[0]) - 1.0
    s1 = 2.0 / (sb[3] - sb[2])
    o1 = -2.0 * sb[2] / (sb[3] - sb[2]) - 1.0
    s2 = 2.0 / (sb[5] - sb[4])
    o2 = -2.0 * sb[4] / (sb[5] - sb[4]) - 1.0
    consts = jnp.stack([s0, o0, s1, o1, s2, o2,
                        jnp.float32(0.0), jnp.float32(0.0)])

    gr = GS_feats.reshape(nbatch, nchan, npts)
    p4 = jnp.transpose(GS_feats[:, :, 0:4], (0, 2, 1))

    Bn = 512
    wpad, idx = pl.pallas_call(
        _pre_body,
        grid=(nbatch, npts // Bn),
        in_specs=[
            pl.BlockSpec(memory_space=pltpu.SMEM),
            pl.BlockSpec((1, C, Bn), lambda b, n: (b, 0, n)),
            pl.BlockSpec((1, 4, Bn), lambda b, n: (b, 0, n)),
        ],
        out_specs=[
            pl.BlockSpec((1, CP, Bn), lambda b, n: (b, 0, n)),
            pl.BlockSpec((1, NPLANES, Bn), lambda b, n: (b, 0, n)),
        ],
        out_shape=[
            jax.ShapeDtypeStruct((nbatch, CP, npts), jnp.float32),
            jax.ShapeDtypeStruct((nbatch, NPLANES, npts), jnp.int32),
        ],
    )(consts, gr, p4)

    wt = jnp.transpose(wpad.reshape(nbatch, 2, HALF, npts), (1, 0, 3, 2))
    idx4 = idx.reshape(nbatch, NPLANES, 1, npts)
    zeros = jnp.zeros((CELLS, HALF), jnp.float32)

    accs = _make_sc_scatter(nbatch, npts)(wt, idx4, zeros)

    Bc = 1024
    nr = nbatch * NPLANES
    a0 = accs[0].reshape(nr, CELLS, HALF)
    a1 = accs[1].reshape(nr, CELLS, HALF)
    o0, o1 = pl.pallas_call(
        _post_body,
        grid=(nr, CELLS // Bc),
        in_specs=[
            pl.BlockSpec((1, Bc, HALF), lambda r, c: (r, c, 0)),
            pl.BlockSpec((1, Bc, HALF), lambda r, c: (r, c, 0)),
        ],
        out_specs=[
            pl.BlockSpec((1, Bc, HALF), lambda r, c: (r, c, 0)),
            pl.BlockSpec((1, Bc, C - HALF), lambda r, c: (r, c, 0)),
        ],
        out_shape=[
            jax.ShapeDtypeStruct((nr, CELLS, HALF), jnp.float32),
            jax.ShapeDtypeStruct((nr, CELLS, C - HALF), jnp.float32),
        ],
    )(a0, a1)

    out = jnp.concatenate([o0, o1], axis=-1)
    out = out.reshape(nbatch, NPLANES, CELLS, C)
    out = jnp.transpose(out, (0, 1, 3, 2))
    return out.reshape(nbatch, NPLANES, C, GRID, GRID)


# speedup vs baseline: 1.1470x; 1.0285x over previous
import functools

import jax
import jax.numpy as jnp
from jax import lax
from jax.experimental import pallas as pl
from jax.experimental.pallas import tpu as pltpu
from jax.experimental.pallas import tpu_sc as plsc

GRID = 128
CELLS = GRID * GRID
C = 196
CP = 208
HALF = CP // 2
NPLANES = 3
NSUB = 16
PCHUNK = 64


def _pre_body(consts_ref, gr_ref, p_ref, w_ref, idx_ref):
    Bn = gr_ref.shape[2]
    npts = pl.num_programs(1) * Bn
    rmod = npts % C
    n0 = pl.program_id(1) * Bn
    a = gr_ref[0]
    ci = lax.broadcasted_iota(jnp.int32, (C, Bn), 0)
    ni = n0 + lax.broadcasted_iota(jnp.int32, (C, Bn), 1)
    q = lax.rem(ni + rmod * ci, C)
    s0 = consts_ref[0]
    o0 = consts_ref[1]
    s1 = consts_ref[2]
    o1 = consts_ref[3]
    s2 = consts_ref[4]
    o2 = consts_ref[5]
    bv = jnp.where(q == 0, a * s0 + o0,
         jnp.where(q == 1, a * s1 + o1,
         jnp.where(q == 2, a * s2 + o2, a)))
    p4 = p_ref[0]
    alpha = jax.nn.sigmoid(p4[3:4, :])
    w_ref[0, 0:C, :] = bv * alpha
    w_ref[0, C:C + 1, :] = alpha
    w_ref[0, C + 1:CP, :] = jnp.zeros((CP - C - 1, Bn), jnp.float32)

    x = p4[0:1, :] * s0 + o0
    y = p4[1:2, :] * s1 + o1
    z = p4[2:3, :] * s2 + o2

    def cell(u):
        g = ((u * 0.5 + 0.5) * (GRID - 1)).astype(jnp.int32)
        return jnp.clip(g, 0, GRID - 1)

    gx, gy, gz = cell(x), cell(y), cell(z)
    idx_ref[0, 0:1, :] = gx * GRID + gy
    idx_ref[0, 1:2, :] = gx * GRID + gz
    idx_ref[0, 2:3, :] = gy * GRID + gz


def _post_body(a0_ref, a1_ref, o0_ref, o1_ref):
    m0 = a0_ref[0]
    m1 = a1_ref[0]
    w = jnp.maximum(m1[:, C - HALF:C - HALF + 1], 1e-6)
    o0_ref[0] = m0 / w
    o1_ref[0] = m1[:, 0:C - HALF] / w


def _make_sc_scatter(nbatch, npts):
    pts_per_sub = npts // NSUB
    nchunks = pts_per_sub // PCHUNK
    rows_per_sub = CELLS // NSUB
    nrounds = nbatch * NPLANES
    mesh = plsc.VectorSubcoreMesh(core_axis_name="c", subcore_axis_name="s")

    npairs = nchunks // 2

    @functools.partial(
        pl.kernel,
        mesh=mesh,
        compiler_params=pltpu.CompilerParams(use_tc_tiling_on_sc=False),
        out_type=jax.ShapeDtypeStruct((2, nbatch, NPLANES, CELLS, HALF),
                                      jnp.float32),
        scratch_types=[
            pltpu.VMEM_SHARED((CELLS, HALF), jnp.float32),
            pltpu.VMEM((PCHUNK,), jnp.int32),
            pltpu.VMEM((PCHUNK, HALF), jnp.float32),
            pltpu.VMEM((PCHUNK,), jnp.int32),
            pltpu.VMEM((PCHUNK, HALF), jnp.float32),
            pltpu.SemaphoreType.DMA,
            pltpu.SemaphoreType.DMA,
        ],
    )
    def sc_scatter(wt_hbm, idx_hbm, z_hbm, out_hbm, acc,
                   idxa, rowsa, idxb, rowsb, sema, semb):
        cid = lax.axis_index("c")
        sid = lax.axis_index("s")
        r0 = sid * rows_per_sub
        pbase = sid * pts_per_sub

        def load(k, idxv, rows, sem, b, p):
            base = pbase + k * PCHUNK
            pltpu.async_copy(idx_hbm.at[b, p, 0, pl.ds(base, PCHUNK)],
                             idxv, sem)
            pltpu.async_copy(wt_hbm.at[cid, b, pl.ds(base, PCHUNK), :],
                             rows, sem)

        def drain(idxv, rows, sem, b, p):
            pltpu.make_async_copy(idx_hbm.at[b, p, 0, pl.ds(pbase, PCHUNK)],
                                  idxv, sem).wait()
            pltpu.make_async_copy(wt_hbm.at[cid, b, pl.ds(pbase, PCHUNK), :],
                                  rows, sem).wait()

        def round_body(r, carry):
            b = r // NPLANES
            p = lax.rem(r, NPLANES)
            pltpu.sync_copy(z_hbm.at[pl.ds(r0, rows_per_sub), :],
                            acc.at[pl.ds(r0, rows_per_sub), :])
            plsc.subcore_barrier()
            load(0, idxa, rowsa, sema, b, p)

            def pair(i, carry2):
                load(2 * i + 1, idxb, rowsb, semb, b, p)
                drain(idxa, rowsa, sema, b, p)
                pltpu.sync_copy(rowsa, acc.at[idxa], add=True)

                @pl.when(i + 1 < npairs)
                def _():
                    load(2 * i + 2, idxa, rowsa, sema, b, p)

                drain(idxb, rowsb, semb, b, p)
                pltpu.sync_copy(rowsb, acc.at[idxb], add=True)
                return carry2

            lax.fori_loop(0, npairs, pair, 0)
            plsc.subcore_barrier()
            pltpu.sync_copy(
                acc.at[pl.ds(r0, rows_per_sub), :],
                out_hbm.at[cid, b, p, pl.ds(r0, rows_per_sub), :])
            return carry

        lax.fori_loop(0, nrounds, round_body, 0)

    return sc_scatter


def kernel(GS_feats, scene_bounds):
    nbatch, npts, nchan = GS_feats.shape
    sb = scene_bounds.astype(jnp.float32)
    s0 = 2.0 / (sb[1] - sb[0])
    o0 = -2.0 * sb[0] / (sb[1] - sb[0]) - 1.0
    s1 = 2.0 / (sb[3] - sb[2])
    o1 = -2.0 * sb[2] / (sb[3] - sb[2]) - 1.0
    s2 = 2.0 / (sb[5] - sb[4])
    o2 = -2.0 * sb[4] / (sb[5] - sb[4]) - 1.0
    consts = jnp.stack([s0, o0, s1, o1, s2, o2,
                        jnp.float32(0.0), jnp.float32(0.0)])

    gr = GS_feats.reshape(nbatch, nchan, npts)
    p4 = jnp.transpose(GS_feats[:, :, 0:4], (0, 2, 1))

    Bn = 512
    wpad, idx = pl.pallas_call(
        _pre_body,
        grid=(nbatch, npts // Bn),
        in_specs=[
            pl.BlockSpec(memory_space=pltpu.SMEM),
            pl.BlockSpec((1, C, Bn), lambda b, n: (b, 0, n)),
            pl.BlockSpec((1, 4, Bn), lambda b, n: (b, 0, n)),
        ],
        out_specs=[
            pl.BlockSpec((1, CP, Bn), lambda b, n: (b, 0, n)),
            pl.BlockSpec((1, NPLANES, Bn), lambda b, n: (b, 0, n)),
        ],
        out_shape=[
            jax.ShapeDtypeStruct((nbatch, CP, npts), jnp.float32),
            jax.ShapeDtypeStruct((nbatch, NPLANES, npts), jnp.int32),
        ],
    )(consts, gr, p4)

    wt = jnp.transpose(wpad.reshape(nbatch, 2, HALF, npts), (1, 0, 3, 2))
    idx4 = idx.reshape(nbatch, NPLANES, 1, npts)
    zeros = jnp.zeros((CELLS, HALF), jnp.float32)

    accs = _make_sc_scatter(nbatch, npts)(wt, idx4, zeros)

    Bc = 1024
    nr = nbatch * NPLANES
    a0 = accs[0].reshape(nr, CELLS, HALF)
    a1 = accs[1].reshape(nr, CELLS, HALF)
    o0, o1 = pl.pallas_call(
        _post_body,
        grid=(nr, CELLS // Bc),
        in_specs=[
            pl.BlockSpec((1, Bc, HALF), lambda r, c: (r, c, 0)),
            pl.BlockSpec((1, Bc, HALF), lambda r, c: (r, c, 0)),
        ],
        out_specs=[
            pl.BlockSpec((1, Bc, HALF), lambda r, c: (r, c, 0)),
            pl.BlockSpec((1, Bc, C - HALF), lambda r, c: (r, c, 0)),
        ],
        out_shape=[
            jax.ShapeDtypeStruct((nr, CELLS, HALF), jnp.float32),
            jax.ShapeDtypeStruct((nr, CELLS, C - HALF), jnp.float32),
        ],
    )(a0, a1)

    out = jnp.concatenate([o0, o1], axis=-1)
    out = out.reshape(nbatch, NPLANES, CELLS, C)
    out = jnp.transpose(out, (0, 1, 3, 2))
    return out.reshape(nbatch, NPLANES, C, GRID, GRID)
